# baseline (speedup 1.0000x reference)
import functools

import jax
import jax.numpy as jnp
from jax import lax
from jax.experimental import pallas as pl
from jax.experimental.pallas import tpu as pltpu
from jax.experimental.pallas import tpu_sc as plsc

N_NODES = 10000
D = 128
NC = 2
NS = 16
NW = NC * NS
CH = 64
OCH = 80
NCH = N_NODES // OCH


def _sc_segment_sum(x, edges, zeros, ncw, rem):
    mesh = plsc.VectorSubcoreMesh(core_axis_name="c", subcore_axis_name="s")
    assert ncw % 6 == 0 and ncw >= 12 and rem < NW

    @functools.partial(
        pl.kernel,
        out_type=jax.ShapeDtypeStruct((NC * N_NODES, D), jnp.float32),
        mesh=mesh,
        scratch_types=[
            pltpu.VMEM_SHARED((N_NODES, D), jnp.float32),
            pltpu.VMEM((6, CH), jnp.int32),
            pltpu.VMEM((6, CH), jnp.int32),
            pltpu.VMEM((3, CH, D), jnp.float32),
            pltpu.SemaphoreType.DMA,
            pltpu.SemaphoreType.DMA,
            pltpu.SemaphoreType.DMA,
            pltpu.SemaphoreType.DMA,
            pltpu.SemaphoreType.DMA,
            pltpu.SemaphoreType.DMA,
            pltpu.SemaphoreType.DMA,
            pltpu.SemaphoreType.DMA,
            pltpu.SemaphoreType.DMA,
            pltpu.SemaphoreType.DMA,
            pltpu.SemaphoreType.DMA,
            pltpu.SemaphoreType.DMA,
            pltpu.SemaphoreType.DMA,
        ],
    )
    def seg_sum(x_hbm, e_hbm, z_hbm, out_hbm, acc, sring, dring, bufs,
                g0, g1, g2, i0, i1, i2, i3, i4, i5, s0, s1, s2, zs):
        gs = (g0, g1, g2)
        iss = (i0, i1, i2, i3, i4, i5)
        ss = (s0, s1, s2)
        c = lax.axis_index("c")
        s = lax.axis_index("s")
        wid = s * NC + c
        base = wid * ncw

        for k0 in range(-(-NCH // NS)):
            k = k0 * NS + s
            @pl.when(k < NCH)
            def _():
                pltpu.async_copy(z_hbm, acc.at[pl.ds(k * OCH, OCH)], zs)
        for k0 in range(-(-NCH // NS)):
            k = k0 * NS + s
            @pl.when(k < NCH)
            def _():
                pltpu.make_async_copy(z_hbm, acc.at[pl.ds(k * OCH, OCH)], zs).wait()
        plsc.subcore_barrier()

        def chunk_off(j):
            return pl.multiple_of((base + lax.rem(j, ncw)) * CH, CH)

        def start_idx(j, q):
            off = chunk_off(j)
            pltpu.async_copy(e_hbm.at[0, pl.ds(off, CH)], sring.at[q], iss[q])
            pltpu.async_copy(e_hbm.at[1, pl.ds(off, CH)], dring.at[q], iss[q])

        def wait_idx(q):
            pltpu.make_async_copy(e_hbm.at[0, pl.ds(0, CH)], sring.at[q],
                                  iss[q]).wait()
            pltpu.make_async_copy(e_hbm.at[1, pl.ds(0, CH)], dring.at[q],
                                  iss[q]).wait()

        def start_gather(t, q):
            pltpu.async_copy(x_hbm.at[sring.at[q]], bufs.at[t], gs[t])

        def wait_gather(t, q):
            pltpu.make_async_copy(x_hbm.at[sring.at[q]], bufs.at[t], gs[t]).wait()

        def start_scatter(t, q):
            pltpu.async_copy(bufs.at[t], acc.at[dring.at[q]], ss[t], add=True)

        def wait_scatter(t, q):
            pltpu.make_async_copy(bufs.at[t], acc.at[dring.at[q]], ss[t]).wait()

        for q in range(4):
            start_idx(q, q)
        wait_idx(0)
        start_gather(0, 0)
        wait_idx(1)
        start_gather(1, 1)

        def slot(j, t, q, first=False):
            wait_gather(t, q)
            if not first:
                wait_scatter((t + 2) % 3, (q + 5) % 6)
            start_scatter(t, q)
            start_idx(j + 4, (q + 4) % 6)
            wait_idx((q + 2) % 6)
            start_gather((t + 2) % 3, (q + 2) % 6)

        slot(0, 0, 0, first=True)
        for j in range(1, 6):
            slot(j, j % 3, j % 6)

        def round_body(r, carry):
            j = 6 * r
            for u in range(6):
                slot(j + u, u % 3, u)
            return carry

        lax.fori_loop(1, ncw // 6, round_body, 0)
        wait_scatter((ncw - 1) % 3, (ncw - 1) % 6)
        wait_gather(0, ncw % 6)
        wait_gather(1, (ncw + 1) % 6)
        wait_idx((ncw + 2) % 6)
        wait_idx((ncw + 3) % 6)

        if rem:
            @pl.when(wid < rem)
            def _():
                off = pl.multiple_of((NW * ncw + wid) * CH, CH)
                pltpu.sync_copy(e_hbm.at[0, pl.ds(off, CH)], sring.at[0])
                pltpu.sync_copy(e_hbm.at[1, pl.ds(off, CH)], dring.at[0])
                start_gather(0, 0)
                wait_gather(0, 0)
                pltpu.sync_copy(bufs.at[0], acc.at[dring.at[0]], add=True)
        plsc.subcore_barrier()

        for k0 in range(-(-NCH // NS)):
            k = k0 * NS + s
            @pl.when(k < NCH)
            def _():
                pltpu.async_copy(acc.at[pl.ds(k * OCH, OCH)],
                                 out_hbm.at[pl.ds(c * N_NODES + k * OCH, OCH)], zs)
        for k0 in range(-(-NCH // NS)):
            k = k0 * NS + s
            @pl.when(k < NCH)
            def _():
                pltpu.make_async_copy(acc.at[pl.ds(k * OCH, OCH)],
                                      out_hbm.at[pl.ds(c * N_NODES + k * OCH, OCH)],
                                      zs).wait()

    return seg_sum(x, edges, zeros)


def _tc_body(x_ref, p0_ref, p1_ref, w_ref, b_ref, out_ref):
    dn = (((1,), (1,)), ((), ()))
    m = p0_ref[...] + p1_ref[...]
    a = lax.dot_general(x_ref[...], w_ref[:, 0:D], dn,
                        preferred_element_type=jnp.float32)
    a = a + lax.dot_general(m, w_ref[:, D:2 * D], dn,
                            preferred_element_type=jnp.float32)
    out_ref[...] = jnp.maximum(a + b_ref[...], 0.0)


def kernel(x, edge_index, W, b):
    n, d = x.shape
    e = edge_index.shape[1]
    assert n == N_NODES and d == D
    assert e % CH == 0

    nchk = e // CH
    ncw = nchk // NW - (nchk // NW) % 6
    rem = nchk - NW * ncw
    edges = edge_index.astype(jnp.int32)
    zeros = jnp.zeros((OCH, D), jnp.float32)

    partials = _sc_segment_sum(x, edges, zeros, ncw, rem)

    b2 = b.reshape(1, D)
    br = 1000
    nb = N_NODES // br
    out = pl.pallas_call(
        _tc_body,
        grid=(nb,),
        in_specs=[
            pl.BlockSpec((br, D), lambda i: (i, 0)),
            pl.BlockSpec((br, D), lambda i: (i, 0)),
            pl.BlockSpec((br, D), lambda i: (i + nb, 0)),
            pl.BlockSpec((D, 2 * D), lambda i: (0, 0)),
            pl.BlockSpec((1, D), lambda i: (0, 0)),
        ],
        out_specs=pl.BlockSpec((br, D), lambda i: (i, 0)),
        out_shape=jax.ShapeDtypeStruct((N_NODES, D), jnp.float32),
    )(x, partials, partials, W, b2)
    return out


# speedup vs baseline: 1.1234x; 1.1234x over previous
import functools

import jax
import jax.numpy as jnp
from jax import lax
from jax.experimental import pallas as pl
from jax.experimental.pallas import tpu as pltpu
from jax.experimental.pallas import tpu_sc as plsc

N_NODES = 10000
D = 128
NC = 2
NS = 16
NW = NC * NS
CH = 128
OCH = 80
NCH = N_NODES // OCH


def _sc_segment_sum(x, edges, zeros, ncw, rem):
    mesh = plsc.VectorSubcoreMesh(core_axis_name="c", subcore_axis_name="s")
    assert ncw % 3 == 0 and ncw >= 6 and rem < NW

    @functools.partial(
        pl.kernel,
        out_type=jax.ShapeDtypeStruct((NC * N_NODES, D), jnp.float32),
        mesh=mesh,
        scratch_types=[
            pltpu.VMEM_SHARED((N_NODES, D), jnp.float32),
            pltpu.VMEM((6, CH), jnp.int32),
            pltpu.VMEM((3, CH, D), jnp.float32),
            pltpu.SemaphoreType.DMA,
            pltpu.SemaphoreType.DMA,
            pltpu.SemaphoreType.DMA,
            pltpu.SemaphoreType.DMA,
            pltpu.SemaphoreType.DMA,
            pltpu.SemaphoreType.DMA,
            pltpu.SemaphoreType.DMA,
        ],
    )
    def seg_sum(x_hbm, e_hbm, z_hbm, out_hbm, acc, ring, bufs,
                g0, g1, g2, i0, i1, i2, zs):
        gs = (g0, g1, g2)
        iss = (i0, i1, i2)
        c = lax.axis_index("c")
        s = lax.axis_index("s")
        wid = s * NC + c
        base = wid * ncw

        def chunk_off(j):
            return pl.multiple_of((base + lax.rem(j, ncw)) * CH, CH)

        def start_idx(j, q):
            off = chunk_off(j)
            pltpu.async_copy(e_hbm.at[0, pl.ds(off, CH)], ring.at[q], iss[q])
            pltpu.async_copy(e_hbm.at[1, pl.ds(off, CH)], ring.at[3 + q], iss[q])

        def wait_idx(q):
            pltpu.make_async_copy(e_hbm.at[0, pl.ds(0, CH)], ring.at[q],
                                  iss[q]).wait()
            pltpu.make_async_copy(e_hbm.at[1, pl.ds(0, CH)], ring.at[3 + q],
                                  iss[q]).wait()

        def start_gather(t):
            pltpu.async_copy(x_hbm.at[ring.at[t]], bufs.at[t], gs[t])

        def wait_gather(t):
            pltpu.make_async_copy(x_hbm.at[ring.at[t]], bufs.at[t], gs[t]).wait()

        pltpu.sync_copy(z_hbm, bufs.at[2].at[pl.ds(0, OCH)])
        for k0 in range(-(-NCH // NS)):
            k = k0 * NS + s
            @pl.when(k < NCH)
            def _():
                pltpu.async_copy(bufs.at[2].at[pl.ds(0, OCH)],
                                 acc.at[pl.ds(k * OCH, OCH)], zs)
        start_idx(0, 0)
        start_idx(1, 1)
        start_idx(2, 2)
        wait_idx(0)
        start_gather(0)
        wait_idx(1)
        start_gather(1)
        for k0 in range(-(-NCH // NS)):
            k = k0 * NS + s
            @pl.when(k < NCH)
            def _():
                pltpu.make_async_copy(bufs.at[2].at[pl.ds(0, OCH)],
                                      acc.at[pl.ds(k * OCH, OCH)], zs).wait()
        plsc.subcore_barrier()

        def slot(j, t):
            wait_gather(t)
            pltpu.sync_copy(bufs.at[t], acc.at[ring.at[3 + t]], add=True)
            start_idx(j + 3, t)
            wait_idx((t + 2) % 3)
            start_gather((t + 2) % 3)

        def round_body(r, carry):
            j = 3 * r
            slot(j, 0)
            slot(j + 1, 1)
            slot(j + 2, 2)
            return carry

        lax.fori_loop(0, ncw // 3, round_body, 0)
        wait_gather(0)
        wait_gather(1)
        wait_idx(2)

        if rem:
            @pl.when(wid < rem)
            def _():
                off = pl.multiple_of((NW * ncw + wid) * CH, CH)
                pltpu.sync_copy(e_hbm.at[0, pl.ds(off, CH)], ring.at[0])
                pltpu.sync_copy(e_hbm.at[1, pl.ds(off, CH)], ring.at[3])
                start_gather(0)
                wait_gather(0)
                pltpu.sync_copy(bufs.at[0], acc.at[ring.at[3]], add=True)
        plsc.subcore_barrier()

        for k0 in range(-(-NCH // NS)):
            k = k0 * NS + s
            @pl.when(k < NCH)
            def _():
                pltpu.async_copy(acc.at[pl.ds(k * OCH, OCH)],
                                 out_hbm.at[pl.ds(c * N_NODES + k * OCH, OCH)], zs)
        for k0 in range(-(-NCH // NS)):
            k = k0 * NS + s
            @pl.when(k < NCH)
            def _():
                pltpu.make_async_copy(acc.at[pl.ds(k * OCH, OCH)],
                                      out_hbm.at[pl.ds(c * N_NODES + k * OCH, OCH)],
                                      zs).wait()

    return seg_sum(x, edges, zeros)


def _tc_body(x_ref, p0_ref, p1_ref, w_ref, b_ref, out_ref):
    dn = (((1,), (1,)), ((), ()))
    m = p0_ref[...] + p1_ref[...]
    a = lax.dot_general(x_ref[...], w_ref[:, 0:D], dn,
                        preferred_element_type=jnp.float32)
    a = a + lax.dot_general(m, w_ref[:, D:2 * D], dn,
                            preferred_element_type=jnp.float32)
    out_ref[...] = jnp.maximum(a + b_ref[...], 0.0)


def kernel(x, edge_index, W, b):
    n, d = x.shape
    e = edge_index.shape[1]
    assert n == N_NODES and d == D
    assert e % CH == 0

    nchk = e // CH
    ncw = nchk // NW - (nchk // NW) % 3
    rem = nchk - NW * ncw
    edges = edge_index.astype(jnp.int32)
    zeros = jnp.zeros((OCH, D), jnp.float32)

    partials = _sc_segment_sum(x, edges, zeros, ncw, rem)

    b2 = b.reshape(1, D)
    br = 1000
    nb = N_NODES // br
    out = pl.pallas_call(
        _tc_body,
        grid=(nb,),
        in_specs=[
            pl.BlockSpec((br, D), lambda i: (i, 0)),
            pl.BlockSpec((br, D), lambda i: (i, 0)),
            pl.BlockSpec((br, D), lambda i: (i + nb, 0)),
            pl.BlockSpec((D, 2 * D), lambda i: (0, 0)),
            pl.BlockSpec((1, D), lambda i: (0, 0)),
        ],
        out_specs=pl.BlockSpec((br, D), lambda i: (i, 0)),
        out_shape=jax.ShapeDtypeStruct((N_NODES, D), jnp.float32),
    )(x, partials, partials, W, b2)
    return out


# speedup vs baseline: 1.1495x; 1.0233x over previous
import functools

import jax
import jax.numpy as jnp
from jax import lax
from jax.experimental import pallas as pl
from jax.experimental.pallas import tpu as pltpu
from jax.experimental.pallas import tpu_sc as plsc

N_NODES = 10000
D = 128
NC = 2
NS = 16
NW = NC * NS
CH = 128
OCH = 80
NCH = N_NODES // OCH


def _sc_segment_sum(x, edges, zeros, ncw, rem):
    mesh = plsc.VectorSubcoreMesh(core_axis_name="c", subcore_axis_name="s")
    assert ncw % 3 == 0 and ncw >= 6 and rem < NW

    @functools.partial(
        pl.kernel,
        out_type=jax.ShapeDtypeStruct((NC * N_NODES, D), jnp.float32),
        mesh=mesh,
        scratch_types=[
            pltpu.VMEM_SHARED((N_NODES, D), jnp.float32),
            pltpu.VMEM((6, CH), jnp.int32),
            pltpu.VMEM((3, CH, D), jnp.float32),
            pltpu.SemaphoreType.DMA,
            pltpu.SemaphoreType.DMA,
            pltpu.SemaphoreType.DMA,
            pltpu.SemaphoreType.DMA,
            pltpu.SemaphoreType.DMA,
            pltpu.SemaphoreType.DMA,
            pltpu.SemaphoreType.DMA,
        ],
    )
    def seg_sum(x_hbm, e_hbm, z_hbm, out_hbm, acc, ring, bufs,
                g0, g1, g2, i0, i1, i2, zs):
        gs = (g0, g1, g2)
        iss = (i0, i1, i2)
        c = lax.axis_index("c")
        s = lax.axis_index("s")
        wid = s * NC + c
        base = wid * ncw

        def chunk_off(j):
            return pl.multiple_of((base + lax.rem(j, ncw)) * CH, CH)

        def start_idx(j, q):
            off = chunk_off(j)
            pltpu.async_copy(e_hbm.at[0, pl.ds(off, CH)], ring.at[q], iss[q])
            pltpu.async_copy(e_hbm.at[1, pl.ds(off, CH)], ring.at[3 + q], iss[q])

        def wait_idx(q):
            pltpu.make_async_copy(e_hbm.at[0, pl.ds(0, CH)], ring.at[q],
                                  iss[q]).wait()
            pltpu.make_async_copy(e_hbm.at[1, pl.ds(0, CH)], ring.at[3 + q],
                                  iss[q]).wait()

        def start_gather(t):
            pltpu.async_copy(x_hbm.at[ring.at[t]], bufs.at[t], gs[t])

        def wait_gather(t):
            pltpu.make_async_copy(x_hbm.at[ring.at[t]], bufs.at[t], gs[t]).wait()

        pltpu.sync_copy(z_hbm, bufs.at[2].at[pl.ds(0, OCH)])
        for k0 in range(-(-NCH // NS)):
            k = k0 * NS + s
            @pl.when(k < NCH)
            def _():
                pltpu.async_copy(bufs.at[2].at[pl.ds(0, OCH)],
                                 acc.at[pl.ds(k * OCH, OCH)], zs)
        start_idx(0, 0)
        start_idx(1, 1)
        start_idx(2, 2)
        wait_idx(0)
        start_gather(0)
        wait_idx(1)
        start_gather(1)
        for k0 in range(-(-NCH // NS)):
            k = k0 * NS + s
            @pl.when(k < NCH)
            def _():
                pltpu.make_async_copy(bufs.at[2].at[pl.ds(0, OCH)],
                                      acc.at[pl.ds(k * OCH, OCH)], zs).wait()
        plsc.subcore_barrier()

        def slot(j, t):
            wait_gather(t)
            pltpu.sync_copy(bufs.at[t], acc.at[ring.at[3 + t]], add=True)
            start_idx(j + 3, t)
            wait_idx((t + 2) % 3)
            start_gather((t + 2) % 3)

        def round_body(r, carry):
            j = 3 * r
            slot(j, 0)
            slot(j + 1, 1)
            slot(j + 2, 2)
            return carry

        lax.fori_loop(0, ncw // 3, round_body, 0)
        wait_gather(0)
        wait_gather(1)
        wait_idx(2)

        if rem:
            @pl.when(wid < rem)
            def _():
                off = pl.multiple_of((NW * ncw + wid) * CH, CH)
                pltpu.sync_copy(e_hbm.at[0, pl.ds(off, CH)], ring.at[0])
                pltpu.sync_copy(e_hbm.at[1, pl.ds(off, CH)], ring.at[3])
                start_gather(0)
                wait_gather(0)
                pltpu.sync_copy(bufs.at[0], acc.at[ring.at[3]], add=True)
        plsc.subcore_barrier()

        for k0 in range(-(-NCH // NS)):
            k = k0 * NS + s
            @pl.when(k < NCH)
            def _():
                pltpu.async_copy(acc.at[pl.ds(k * OCH, OCH)],
                                 out_hbm.at[pl.ds(c * N_NODES + k * OCH, OCH)], zs)
        for k0 in range(-(-NCH // NS)):
            k = k0 * NS + s
            @pl.when(k < NCH)
            def _():
                pltpu.make_async_copy(acc.at[pl.ds(k * OCH, OCH)],
                                      out_hbm.at[pl.ds(c * N_NODES + k * OCH, OCH)],
                                      zs).wait()

    return seg_sum(x, edges, zeros)


def _tc_body(x_ref, p0_ref, p1_ref, w_ref, b_ref, out_ref):
    dn = (((1,), (1,)), ((), ()))
    m = p0_ref[...] + p1_ref[...]
    a = lax.dot_general(x_ref[...], w_ref[:, 0:D], dn,
                        preferred_element_type=jnp.float32)
    a = a + lax.dot_general(m, w_ref[:, D:2 * D], dn,
                            preferred_element_type=jnp.float32)
    out_ref[...] = jnp.maximum(a + b_ref[...], 0.0)


def kernel(x, edge_index, W, b):
    n, d = x.shape
    e = edge_index.shape[1]
    assert n == N_NODES and d == D
    assert e % CH == 0

    nchk = e // CH
    ncw = nchk // NW - (nchk // NW) % 3
    rem = nchk - NW * ncw
    edges = edge_index.astype(jnp.int32)
    zeros = jnp.zeros((OCH, D), jnp.float32)

    partials = _sc_segment_sum(x, edges, zeros, ncw, rem)

    b2 = b.reshape(1, D)
    br = 2000
    nb = N_NODES // br
    out = pl.pallas_call(
        _tc_body,
        grid=(nb,),
        in_specs=[
            pl.BlockSpec((br, D), lambda i: (i, 0)),
            pl.BlockSpec((br, D), lambda i: (i, 0)),
            pl.BlockSpec((br, D), lambda i: (i + nb, 0)),
            pl.BlockSpec((D, 2 * D), lambda i: (0, 0)),
            pl.BlockSpec((1, D), lambda i: (0, 0)),
        ],
        out_specs=pl.BlockSpec((br, D), lambda i: (i, 0)),
        out_shape=jax.ShapeDtypeStruct((N_NODES, D), jnp.float32),
    )(x, partials, partials, W, b2)
    return out
